# baseline (speedup 1.0000x reference)
import functools

import jax
import jax.numpy as jnp
from jax import lax
from jax.experimental import pallas as pl
from jax.experimental.pallas import tpu as pltpu
from jax.experimental.pallas import tpu_sc as plsc

NC = 2
NS = 16
NW = NC * NS
K = 8


def _emb_body(ids_hbm, table_hbm, out_hbm, idx_v, rows_v, spm,
              gsem0, gsem1, tsem0, tsem1, ssem0, ssem1):
    cpw = ids_hbm.shape[0] // NW
    sid = lax.axis_index("s")
    wid = sid * NC + lax.axis_index("c")
    chunk0 = wid * cpw
    pltpu.sync_copy(ids_hbm.at[pl.ds(chunk0 * 1, cpw)], idx_v)
    gsems = (gsem0, gsem1)
    tsems = (tsem0, tsem1)
    ssems = (ssem0, ssem1)

    def gather(g, b):
        pltpu.async_copy(table_hbm.at[idx_v.at[g]], rows_v.at[b], gsems[b])

    def wait_gather(b):
        pltpu.make_async_copy(
            table_hbm.at[idx_v.at[0]], rows_v.at[b], gsems[b]).wait()

    def t2s(b):
        pltpu.async_copy(rows_v.at[b], spm.at[sid, b], tsems[b])

    def wait_t2s(b):
        pltpu.make_async_copy(rows_v.at[b], spm.at[sid, b], tsems[b]).wait()

    def s2h(g, b):
        pltpu.async_copy(
            spm.at[sid, b], out_hbm.at[pl.ds((chunk0 + g) * K, K)], ssems[b])

    def wait_s2h(b):
        pltpu.make_async_copy(
            spm.at[sid, b], out_hbm.at[pl.ds(chunk0 * K, K)], ssems[b]).wait()

    gather(0, 0)
    gather(1, 1)

    for j in range(2):
        b = j % 2
        wait_gather(b)
        t2s(b)
        wait_t2s(b)
        s2h(j, b)
        gather(j + 2, b)

    def step(h, _):
        for b in range(2):
            j = h * 2 + b + 2
            wait_gather(b)
            wait_s2h(b)
            t2s(b)
            wait_t2s(b)
            s2h(j, b)
            gather(j + 2, b)
        return 0

    lax.fori_loop(0, (cpw - 4) // 2, step, 0)

    for j in range(cpw - 2, cpw):
        b = j % 2
        wait_gather(b)
        wait_s2h(b)
        t2s(b)
        wait_t2s(b)
        s2h(j, b)

    wait_s2h(0)
    wait_s2h(1)


def _make_emb(n_tokens, vocab, d_model):
    mesh = plsc.VectorSubcoreMesh(core_axis_name="c", subcore_axis_name="s")
    return functools.partial(
        pl.kernel,
        mesh=mesh,
        out_type=jax.ShapeDtypeStruct((n_tokens, d_model), jnp.float32),
        scratch_types=[
            pltpu.VMEM((n_tokens // K // NW, K), jnp.int32),
            pltpu.VMEM((2, K, d_model), jnp.float32),
            pltpu.VMEM_SHARED((NS, 2, K, d_model), jnp.float32),
        ] + [pltpu.SemaphoreType.DMA] * 6,
    )(_emb_body)


def kernel(input_ids, attention_mask, labels, weight):
    b, s = input_ids.shape
    vocab, d_model = weight.shape
    ids2d = input_ids.reshape(-1, K).astype(jnp.int32)
    out = _make_emb(b * s, vocab, d_model)(ids2d, weight)
    hidden_states = out.reshape(b, s, d_model)
    position_ids = jnp.arange(s, dtype=jnp.int32)[None, :]
    return (hidden_states, attention_mask, position_ids, labels)


# speedup vs baseline: 1.0052x; 1.0052x over previous
import functools

import jax
import jax.numpy as jnp
from jax import lax
from jax.experimental import pallas as pl
from jax.experimental.pallas import tpu as pltpu
from jax.experimental.pallas import tpu_sc as plsc

NC = 2
NS = 16
NW = NC * NS
K = 8


NBUF = 4


def _emb_body(ids_hbm, table_hbm, out_hbm, idx_v, rows_v,
              gsem0, gsem1, gsem2, gsem3):
    tpw = ids_hbm.shape[0] // NW
    cpw = tpw // K
    wid = lax.axis_index("s") * NC + lax.axis_index("c")
    chunk0 = wid * cpw
    pltpu.sync_copy(ids_hbm.at[pl.ds(wid * tpw, tpw)], idx_v)
    gsems = (gsem0, gsem1, gsem2, gsem3)

    def gather(g, b):
        pltpu.async_copy(table_hbm.at[idx_v.at[pl.ds(g * K, K)]], rows_v.at[b], gsems[b])

    def wait_gather(b):
        pltpu.make_async_copy(
            table_hbm.at[idx_v.at[pl.ds(0, K)]], rows_v.at[b], gsems[b]).wait()

    def scatter(g, b):
        pltpu.sync_copy(rows_v.at[b], out_hbm.at[pl.ds((chunk0 + g) * K, K)])

    for b in range(NBUF):
        gather(b, b)

    def step(h, _):
        for b in range(NBUF):
            j = h * NBUF + b
            wait_gather(b)
            scatter(j, b)
            gather(j + NBUF, b)
        return 0

    lax.fori_loop(0, (cpw - NBUF) // NBUF, step, 0)

    for j in range(cpw - NBUF, cpw):
        b = j % NBUF
        wait_gather(b)
        scatter(j, b)


def _make_emb(n_tokens, vocab, d_model):
    mesh = plsc.VectorSubcoreMesh(core_axis_name="c", subcore_axis_name="s")
    return functools.partial(
        pl.kernel,
        mesh=mesh,
        out_type=jax.ShapeDtypeStruct((n_tokens, d_model), jnp.float32),
        scratch_types=[
            pltpu.VMEM((n_tokens // NW,), jnp.int32),
            pltpu.VMEM((NBUF, K, d_model), jnp.float32),
        ] + [pltpu.SemaphoreType.DMA] * NBUF,
    )(_emb_body)


def kernel(input_ids, attention_mask, labels, weight):
    b, s = input_ids.shape
    vocab, d_model = weight.shape
    ids_flat = input_ids.reshape(-1).astype(jnp.int32)
    out = _make_emb(b * s, vocab, d_model)(ids_flat, weight)
    hidden_states = out.reshape(b, s, d_model)
    position_ids = jnp.arange(s, dtype=jnp.int32)[None, :]
    return (hidden_states, attention_mask, position_ids, labels)
